# Initial kernel scaffold; baseline (speedup 1.0000x reference)
import functools

import jax
import jax.numpy as jnp
from jax import lax
from jax.experimental import pallas as pl
from jax.experimental.pallas import tpu as pltpu
from jax.experimental.pallas import tpu_sc as plsc

N = 10000
D = 128
EPS = 0.1
NC = 2
NS = 16
NW = NC * NS
CHUNK = 128
ROWS_PER_TILE = N // NS
ZROWS = 125


def _alpha_body(x_ref, a_ref, o_ref):
    o_ref[...] = jnp.dot(x_ref[...], a_ref[...],
                         preferred_element_type=jnp.float32)


def _alpha_call(x, a2):
    return pl.pallas_call(
        _alpha_body,
        out_shape=jax.ShapeDtypeStruct((N, 128), jnp.float32),
        grid=(8,),
        in_specs=[
            pl.BlockSpec((N // 8, D), lambda i: (i, 0)),
            pl.BlockSpec((D, 128), lambda i: (0, 0)),
        ],
        out_specs=pl.BlockSpec((N // 8, 128), lambda i: (i, 0)),
    )(x, a2)


def _sc_body(nch, x_hbm, al_hbm, ar_hbm, src_hbm, dst_hbm, w_hbm, out_hbm,
             al_v, ar_v, src_v, dst_v, w_v, coef_v, rows_v, acc):
    cid = lax.axis_index("c")
    sid = lax.axis_index("s")
    wid = cid * NS + sid

    pltpu.sync_copy(al_hbm, al_v)
    pltpu.sync_copy(ar_hbm, ar_v)
    pltpu.sync_copy(src_hbm.at[wid], src_v)
    pltpu.sync_copy(dst_hbm.at[wid], dst_v)
    pltpu.sync_copy(w_hbm.at[wid], w_v)

    zeros16 = jnp.zeros((16,), jnp.float32)

    def zrow(i, carry):
        for j in range(8):
            rows_v[i, pl.ds(j * 16, 16)] = zeros16
        return carry

    lax.fori_loop(0, ZROWS, zrow, 0)
    base = sid * ROWS_PER_TILE
    for k in range(ROWS_PER_TILE // ZROWS):
        pltpu.sync_copy(rows_v.at[pl.ds(0, ZROWS)],
                        acc.at[pl.ds(base + k * ZROWS, ZROWS)])
    plsc.subcore_barrier()

    def chunk_body(c, carry):
        pltpu.sync_copy(x_hbm.at[src_v.at[c]], rows_v)
        for j in range(8):
            sv = src_v[c, pl.ds(j * 16, 16)]
            dv = dst_v[c, pl.ds(j * 16, 16)]
            wv = w_v[c, pl.ds(j * 16, 16)]
            al = plsc.load_gather(al_v, [sv])
            ar = plsc.load_gather(ar_v, [dv])
            t = 1.0 - 2.0 / (1.0 + jnp.exp(2.0 * (al + ar)))
            coef_v[pl.ds(j * 16, 16)] = t * wv

        def scale_row(i, carry2):
            cs = coef_v[i]
            for j in range(8):
                sl = pl.ds(j * 16, 16)
                rows_v[i, sl] = rows_v[i, sl] * cs
            return carry2

        lax.fori_loop(0, CHUNK, scale_row, 0)
        pltpu.sync_copy(rows_v, acc.at[dst_v.at[c]], add=True)
        return carry

    lax.fori_loop(0, nch, chunk_body, 0)

    plsc.subcore_barrier()
    pltpu.sync_copy(acc.at[pl.ds(base, ROWS_PER_TILE)],
                    out_hbm.at[pl.ds(cid * N + base, ROWS_PER_TILE)])


def _sc_call(nch, x, al, ar, srcs, dsts, ws):
    mesh = plsc.VectorSubcoreMesh(core_axis_name="c", subcore_axis_name="s")
    return pl.kernel(
        functools.partial(_sc_body, nch),
        out_type=jax.ShapeDtypeStruct((NC * N, D), jnp.float32),
        mesh=mesh,
        scratch_types=[
            pltpu.VMEM((N,), jnp.float32),
            pltpu.VMEM((N,), jnp.float32),
            pltpu.VMEM((nch, CHUNK), jnp.int32),
            pltpu.VMEM((nch, CHUNK), jnp.int32),
            pltpu.VMEM((nch, CHUNK), jnp.float32),
            pltpu.VMEM((CHUNK,), jnp.float32),
            pltpu.VMEM((CHUNK, D), jnp.float32),
            pltpu.VMEM_SHARED((N, D), jnp.float32),
        ],
    )(x, al, ar, srcs, dsts, ws)


def _out_body(p0_ref, p1_ref, x0_ref, w_ref, b_ref, o_ref):
    s = p0_ref[...] + p1_ref[...] + EPS * x0_ref[...]
    h = jnp.maximum(s, 0.0)
    o_ref[...] = jnp.dot(h, w_ref[...],
                         preferred_element_type=jnp.float32) + b_ref[...]


def _out_call(parts, x_0, W, b2):
    blk = N // 10
    return pl.pallas_call(
        _out_body,
        out_shape=jax.ShapeDtypeStruct((N, D), jnp.float32),
        grid=(10,),
        in_specs=[
            pl.BlockSpec((None, blk, D), lambda i: (0, i, 0)),
            pl.BlockSpec((None, blk, D), lambda i: (1, i, 0)),
            pl.BlockSpec((blk, D), lambda i: (i, 0)),
            pl.BlockSpec((D, D), lambda i: (0, 0)),
            pl.BlockSpec((1, D), lambda i: (0, 0)),
        ],
        out_specs=pl.BlockSpec((blk, D), lambda i: (i, 0)),
    )(parts, parts, x_0, W, b2)


def kernel(x, x_0, edge_index, edge_weight, a_l, a_r, W, b):
    e = edge_index.shape[1]
    nch = -(-e // (NW * CHUNK))
    pad = NW * nch * CHUNK - e
    src = edge_index[0].astype(jnp.int32)
    dst = edge_index[1].astype(jnp.int32)
    ew = edge_weight.astype(jnp.float32)
    if pad:
        zi = jnp.zeros((pad,), jnp.int32)
        src = jnp.concatenate([src, zi])
        dst = jnp.concatenate([dst, zi])
        ew = jnp.concatenate([ew, jnp.zeros((pad,), jnp.float32)])
    srcs = src.reshape(NW, nch, CHUNK)
    dsts = dst.reshape(NW, nch, CHUNK)
    ws = ew.reshape(NW, nch, CHUNK)

    a2 = jnp.zeros((D, 128), jnp.float32)
    a2 = a2.at[:, 0].set(a_l).at[:, 1].set(a_r)
    alphas = _alpha_call(x, a2)
    al = alphas[:, 0]
    ar = alphas[:, 1]

    parts = _sc_call(nch, x, al, ar, srcs, dsts, ws).reshape(2, N, D)
    return _out_call(parts, x_0, W, b2=b.reshape(1, D))


# speedup vs baseline: 8.5899x; 8.5899x over previous
import functools

import jax
import jax.numpy as jnp
from jax import lax
from jax.experimental import pallas as pl
from jax.experimental.pallas import tpu as pltpu
from jax.experimental.pallas import tpu_sc as plsc

N = 10000
D = 128
EPS = 0.1
NC = 2
NS = 16
NW = NC * NS
CHUNK = 128
GS = 8
NP = 10240
ROWS_PER_TILE = NP // NS
ZROWS = 128


def _alpha_body(x_ref, a_ref, o_ref):
    o_ref[...] = jnp.dot(x_ref[...], a_ref[...],
                         preferred_element_type=jnp.float32)


def _alpha_call(x, a2):
    return pl.pallas_call(
        _alpha_body,
        out_shape=jax.ShapeDtypeStruct((N, 128), jnp.float32),
        grid=(10,),
        in_specs=[
            pl.BlockSpec((N // 10, D), lambda i: (i, 0)),
            pl.BlockSpec((D, 128), lambda i: (0, 0)),
        ],
        out_specs=pl.BlockSpec((N // 10, 128), lambda i: (i, 0)),
    )(x, a2)


def _sc_body(nch, x_hbm, al_hbm, ar_hbm, src_hbm, dst_hbm, w_hbm, out_hbm,
             al_v, ar_v, src_v, dst_v, w_v, coef_v, rows_v, acc):
    cid = lax.axis_index("c")
    sid = lax.axis_index("s")
    wid = cid * NS + sid

    pltpu.sync_copy(al_hbm, al_v)
    pltpu.sync_copy(ar_hbm, ar_v)

    zeros16 = jnp.zeros((16,), jnp.float32)

    def zrow(i, carry):
        for j in range(8):
            rows_v[i, pl.ds(j * 16, 16)] = zeros16
        return carry

    lax.fori_loop(0, ZROWS, zrow, 0)
    base = sid * ROWS_PER_TILE
    for k in range(ROWS_PER_TILE // ZROWS):
        pltpu.sync_copy(rows_v.at[pl.ds(0, ZROWS)],
                        acc.at[pl.ds(base + k * ZROWS, ZROWS)])
    plsc.subcore_barrier()

    def group_body(g, carry):
        gsl = pl.ds(g * GS, GS)
        pltpu.sync_copy(src_hbm.at[wid, gsl], src_v)
        pltpu.sync_copy(dst_hbm.at[wid, gsl], dst_v)
        pltpu.sync_copy(w_hbm.at[wid, gsl], w_v)

        def chunk_body(c, carry1):
            pltpu.sync_copy(x_hbm.at[src_v.at[c]], rows_v)
            for j in range(8):
                sv = src_v[c, pl.ds(j * 16, 16)]
                dv = dst_v[c, pl.ds(j * 16, 16)]
                wv = w_v[c, pl.ds(j * 16, 16)]
                al = plsc.load_gather(al_v, [sv])
                ar = plsc.load_gather(ar_v, [dv])
                t = 1.0 - 2.0 / (1.0 + jnp.exp(2.0 * (al + ar)))
                coef_v[pl.ds(j * 16, 16)] = t * wv

            def scale_row(i, carry2):
                cs = plsc.load_gather(coef_v, [jnp.full((16,), i, jnp.int32)])
                for j in range(8):
                    sl = pl.ds(j * 16, 16)
                    rows_v[i, sl] = rows_v[i, sl] * cs
                return carry2

            lax.fori_loop(0, CHUNK, scale_row, 0)
            pltpu.sync_copy(rows_v, acc.at[dst_v.at[c]], add=True)
            return carry1

        lax.fori_loop(0, GS, chunk_body, 0)
        return carry

    lax.fori_loop(0, nch // GS, group_body, 0)

    plsc.subcore_barrier()
    pltpu.sync_copy(acc.at[pl.ds(base, ROWS_PER_TILE)],
                    out_hbm.at[pl.ds(cid * NP + base, ROWS_PER_TILE)])


def _sc_call(nch, x, al, ar, srcs, dsts, ws):
    mesh = plsc.VectorSubcoreMesh(core_axis_name="c", subcore_axis_name="s")
    return pl.kernel(
        functools.partial(_sc_body, nch),
        out_type=jax.ShapeDtypeStruct((NC * NP, D), jnp.float32),
        mesh=mesh,
        compiler_params=pltpu.CompilerParams(needs_layout_passes=False),
        scratch_types=[
            pltpu.VMEM((N,), jnp.float32),
            pltpu.VMEM((N,), jnp.float32),
            pltpu.VMEM((GS, CHUNK), jnp.int32),
            pltpu.VMEM((GS, CHUNK), jnp.int32),
            pltpu.VMEM((GS, CHUNK), jnp.float32),
            pltpu.VMEM((CHUNK,), jnp.float32),
            pltpu.VMEM((CHUNK, D), jnp.float32),
            pltpu.VMEM_SHARED((NP, D), jnp.float32),
        ],
    )(x, al, ar, srcs, dsts, ws)


def _out_body(p0_ref, p1_ref, x0_ref, w_ref, b_ref, o_ref):
    s = p0_ref[...] + p1_ref[...] + EPS * x0_ref[...]
    h = jnp.maximum(s, 0.0)
    o_ref[...] = jnp.dot(h, w_ref[...],
                         preferred_element_type=jnp.float32) + b_ref[...]


def _out_call(parts, x_0, W, b2):
    blk = N // 10
    return pl.pallas_call(
        _out_body,
        out_shape=jax.ShapeDtypeStruct((N, D), jnp.float32),
        grid=(10,),
        in_specs=[
            pl.BlockSpec((None, blk, D), lambda i: (0, i, 0)),
            pl.BlockSpec((None, blk, D), lambda i: (1, i, 0)),
            pl.BlockSpec((blk, D), lambda i: (i, 0)),
            pl.BlockSpec((D, D), lambda i: (0, 0)),
            pl.BlockSpec((1, D), lambda i: (0, 0)),
        ],
        out_specs=pl.BlockSpec((blk, D), lambda i: (i, 0)),
    )(parts, parts, x_0, W, b2)


def kernel(x, x_0, edge_index, edge_weight, a_l, a_r, W, b):
    e = edge_index.shape[1]
    nch = -(-e // (NW * CHUNK))
    nch = -(-nch // GS) * GS
    pad = NW * nch * CHUNK - e
    src = edge_index[0].astype(jnp.int32)
    dst = edge_index[1].astype(jnp.int32)
    ew = edge_weight.astype(jnp.float32)
    if pad:
        zi = jnp.zeros((pad,), jnp.int32)
        src = jnp.concatenate([src, zi])
        dst = jnp.concatenate([dst, zi])
        ew = jnp.concatenate([ew, jnp.zeros((pad,), jnp.float32)])
    srcs = src.reshape(NW, nch, CHUNK)
    dsts = dst.reshape(NW, nch, CHUNK)
    ws = ew.reshape(NW, nch, CHUNK)

    a2 = jnp.zeros((D, 128), jnp.float32)
    a2 = a2.at[:, 0].set(a_l).at[:, 1].set(a_r)
    alphas = _alpha_call(x, a2)
    al = alphas[:, 0]
    ar = alphas[:, 1]

    parts = _sc_call(nch, x, al, ar, srcs, dsts, ws).reshape(2, NP, D)
    return _out_call(parts, x_0, W, b2=b.reshape(1, D))


# speedup vs baseline: 10.7169x; 1.2476x over previous
import functools

import jax
import jax.numpy as jnp
from jax import lax
from jax.experimental import pallas as pl
from jax.experimental.pallas import tpu as pltpu
from jax.experimental.pallas import tpu_sc as plsc

N = 10000
D = 128
EPS = 0.1
NC = 2
NS = 16
NW = NC * NS
CHUNK = 128
GS = 8
NP = 10240
ROWS_PER_TILE = NP // NS
ZROWS = 128


def _alpha_body(x_ref, a_ref, o_ref):
    o_ref[...] = jnp.dot(x_ref[...], a_ref[...],
                         preferred_element_type=jnp.float32)


def _alpha_call(x, a2):
    return pl.pallas_call(
        _alpha_body,
        out_shape=jax.ShapeDtypeStruct((N, 128), jnp.float32),
        grid=(10,),
        in_specs=[
            pl.BlockSpec((N // 10, D), lambda i: (i, 0)),
            pl.BlockSpec((D, 128), lambda i: (0, 0)),
        ],
        out_specs=pl.BlockSpec((N // 10, 128), lambda i: (i, 0)),
    )(x, a2)


def _sc_mesh():
    return plsc.VectorSubcoreMesh(core_axis_name="c", subcore_axis_name="s")


def _coef_body(nch, al_hbm, ar_hbm, src_hbm, dst_hbm, w_hbm, cf_hbm,
               al_v, ar_v, srcg, dstg, wg, cfg):
    cid = lax.axis_index("c")
    sid = lax.axis_index("s")
    wid = cid * NS + sid

    pltpu.sync_copy(al_hbm, al_v)
    pltpu.sync_copy(ar_hbm, ar_v)

    def group_body(g, carry):
        gsl = pl.ds(g * GS, GS)
        pltpu.sync_copy(src_hbm.at[wid, gsl], srcg)
        pltpu.sync_copy(dst_hbm.at[wid, gsl], dstg)
        pltpu.sync_copy(w_hbm.at[wid, gsl], wg)
        for k in range(GS):
            for j in range(8):
                sl = pl.ds(j * 16, 16)
                sv = srcg[k, sl]
                dv = dstg[k, sl]
                wv = wg[k, sl]
                al = plsc.load_gather(al_v, [sv])
                ar = plsc.load_gather(ar_v, [dv])
                t = 1.0 - 2.0 / (1.0 + jnp.exp(2.0 * (al + ar)))
                cfg[k, sl] = t * wv
        pltpu.sync_copy(cfg, cf_hbm.at[wid, gsl])
        return carry

    lax.fori_loop(0, nch // GS, group_body, 0)


def _coef_call(nch, al, ar, srcs, dsts, ws):
    return pl.kernel(
        functools.partial(_coef_body, nch),
        out_type=jax.ShapeDtypeStruct((NW, nch, CHUNK), jnp.float32),
        mesh=_sc_mesh(),
        compiler_params=pltpu.CompilerParams(needs_layout_passes=False),
        scratch_types=[
            pltpu.VMEM((N,), jnp.float32),
            pltpu.VMEM((N,), jnp.float32),
            pltpu.VMEM((GS, CHUNK), jnp.int32),
            pltpu.VMEM((GS, CHUNK), jnp.int32),
            pltpu.VMEM((GS, CHUNK), jnp.float32),
            pltpu.VMEM((GS, CHUNK), jnp.float32),
        ],
    )(al, ar, srcs, dsts, ws)


def _sc_body(nch, x_hbm, src_hbm, dst_hbm, cf_hbm, out_hbm,
             src_v, dst_v, cf_v, rows0, rows1, acc, sg0, sg1, si):
    cid = lax.axis_index("c")
    sid = lax.axis_index("s")
    wid = cid * NS + sid
    ngroups = nch // GS

    pltpu.sync_copy(src_hbm.at[wid], src_v)

    zeros16 = jnp.zeros((16,), jnp.float32)

    def zrow(i, carry):
        for j in range(8):
            rows0[i, pl.ds(j * 16, 16)] = zeros16
        return carry

    lax.fori_loop(0, ZROWS, zrow, 0)
    base = sid * ROWS_PER_TILE
    for k in range(ROWS_PER_TILE // ZROWS):
        pltpu.sync_copy(rows0.at[pl.ds(0, ZROWS)],
                        acc.at[pl.ds(base + k * ZROWS, ZROWS)])
    plsc.subcore_barrier()

    pltpu.sync_copy(dst_hbm.at[wid, pl.ds(0, GS)], dst_v.at[0])
    pltpu.sync_copy(cf_hbm.at[wid, pl.ds(0, GS)], cf_v.at[0])
    pltpu.async_copy(x_hbm.at[src_v.at[0]], rows0, sg0)

    bufs = (rows0, rows1)
    sems = (sg0, sg1)

    def group_body(g, carry):
        p = lax.rem(g, 2)
        @pl.when(g + 1 < ngroups)
        def _():
            nsl = pl.ds((g + 1) * GS, GS)
            pltpu.async_copy(dst_hbm.at[wid, nsl], dst_v.at[1 - p], si)
            pltpu.async_copy(cf_hbm.at[wid, nsl], cf_v.at[1 - p], si)

        for k in range(GS):
            c = g * GS + k
            buf = bufs[k % 2]
            sem = sems[k % 2]
            nbuf = bufs[(k + 1) % 2]
            nsem = sems[(k + 1) % 2]
            if k + 1 < GS:
                pltpu.async_copy(x_hbm.at[src_v.at[c + 1]], nbuf, nsem)
            else:
                @pl.when(g + 1 < ngroups)
                def _():
                    pltpu.async_copy(x_hbm.at[src_v.at[c + 1]], nbuf, nsem)
            pltpu.make_async_copy(x_hbm.at[src_v.at[c]], buf, sem).wait()

            def scale_row(i, carry2):
                cs = plsc.load_gather(
                    cf_v, [jnp.full((16,), p, jnp.int32),
                           jnp.full((16,), k, jnp.int32),
                           jnp.full((16,), i, jnp.int32)])
                for j in range(8):
                    sl = pl.ds(j * 16, 16)
                    buf[i, sl] = buf[i, sl] * cs
                return carry2

            lax.fori_loop(0, CHUNK, scale_row, 0)
            pltpu.sync_copy(buf, acc.at[dst_v.at[p, k]], add=True)

        @pl.when(g + 1 < ngroups)
        def _():
            pltpu.make_async_copy(dst_hbm.at[wid, pl.ds(0, GS)],
                                  dst_v.at[1 - p], si).wait()
            pltpu.make_async_copy(cf_hbm.at[wid, pl.ds(0, GS)],
                                  cf_v.at[1 - p], si).wait()
        return carry

    lax.fori_loop(0, ngroups, group_body, 0)

    plsc.subcore_barrier()
    pltpu.sync_copy(acc.at[pl.ds(base, ROWS_PER_TILE)],
                    out_hbm.at[pl.ds(cid * NP + base, ROWS_PER_TILE)])


def _sc_call(nch, x, srcs, dsts, cfs):
    return pl.kernel(
        functools.partial(_sc_body, nch),
        out_type=jax.ShapeDtypeStruct((NC * NP, D), jnp.float32),
        mesh=_sc_mesh(),
        compiler_params=pltpu.CompilerParams(needs_layout_passes=False),
        scratch_types=[
            pltpu.VMEM((nch, CHUNK), jnp.int32),
            pltpu.VMEM((2, GS, CHUNK), jnp.int32),
            pltpu.VMEM((2, GS, CHUNK), jnp.float32),
            pltpu.VMEM((CHUNK, D), jnp.float32),
            pltpu.VMEM((CHUNK, D), jnp.float32),
            pltpu.VMEM_SHARED((NP, D), jnp.float32),
            pltpu.SemaphoreType.DMA,
            pltpu.SemaphoreType.DMA,
            pltpu.SemaphoreType.DMA,
        ],
    )(x, srcs, dsts, cfs)


def _out_body(p0_ref, p1_ref, x0_ref, w_ref, b_ref, o_ref):
    s = p0_ref[...] + p1_ref[...] + EPS * x0_ref[...]
    h = jnp.maximum(s, 0.0)
    o_ref[...] = jnp.dot(h, w_ref[...],
                         preferred_element_type=jnp.float32) + b_ref[...]


def _out_call(parts, x_0, W, b2):
    blk = N // 10
    return pl.pallas_call(
        _out_body,
        out_shape=jax.ShapeDtypeStruct((N, D), jnp.float32),
        grid=(10,),
        in_specs=[
            pl.BlockSpec((None, blk, D), lambda i: (0, i, 0)),
            pl.BlockSpec((None, blk, D), lambda i: (1, i, 0)),
            pl.BlockSpec((blk, D), lambda i: (i, 0)),
            pl.BlockSpec((D, D), lambda i: (0, 0)),
            pl.BlockSpec((1, D), lambda i: (0, 0)),
        ],
        out_specs=pl.BlockSpec((blk, D), lambda i: (i, 0)),
    )(parts, parts, x_0, W, b2)


def kernel(x, x_0, edge_index, edge_weight, a_l, a_r, W, b):
    e = edge_index.shape[1]
    nch = -(-e // (NW * CHUNK))
    nch = -(-nch // GS) * GS
    pad = NW * nch * CHUNK - e
    src = edge_index[0].astype(jnp.int32)
    dst = edge_index[1].astype(jnp.int32)
    ew = edge_weight.astype(jnp.float32)
    if pad:
        zi = jnp.zeros((pad,), jnp.int32)
        src = jnp.concatenate([src, zi])
        dst = jnp.concatenate([dst, zi])
        ew = jnp.concatenate([ew, jnp.zeros((pad,), jnp.float32)])
    srcs = src.reshape(NW, nch, CHUNK)
    dsts = dst.reshape(NW, nch, CHUNK)
    ws = ew.reshape(NW, nch, CHUNK)

    a2 = jnp.zeros((D, 128), jnp.float32)
    a2 = a2.at[:, 0].set(a_l).at[:, 1].set(a_r)
    alphas = _alpha_call(x, a2)
    al = alphas[:, 0]
    ar = alphas[:, 1]

    cfs = _coef_call(nch, al, ar, srcs, dsts, ws)
    parts = _sc_call(nch, x, srcs, dsts, cfs).reshape(2, NP, D)
    return _out_call(parts, x_0, W, b2=b.reshape(1, D))
